# baseline (speedup 1.0000x reference)
import functools

import jax
import jax.numpy as jnp
from jax import lax
from jax.experimental import pallas as pl
from jax.experimental.pallas import tpu as pltpu
from jax.experimental.pallas import tpu_sc as plsc

N_NUM = 13
N_CAT = 26
CARD = 100000
D = 64
BATCH = 4096
N_TOK = N_NUM + N_CAT

NC = 2
NS = 16
NW = NC * NS
B_PER_W = BATCH // NW
CAT_ROWS_W = B_PER_W * N_CAT
NUM_ROWS_W = B_PER_W * N_NUM
CHUNK = 128
CAT_CHUNKS = CAT_ROWS_W // CHUNK
NUM_CHUNKS = NUM_ROWS_W // CHUNK
L = 16


def _num_tokens_tc(x_num, num_weight, num_bias, ln_gamma, ln_beta):
    def body(x_ref, w_ref, b_ref, g_ref, be_ref, o_ref):
        x = x_ref[...]
        t = x[:, :, None] * w_ref[...][None] + b_ref[...][None]
        mu = jnp.mean(t, axis=-1, keepdims=True)
        var = jnp.mean((t - mu) * (t - mu), axis=-1, keepdims=True)
        t = (t - mu) / jnp.sqrt(var + 1e-5)
        o_ref[...] = t * g_ref[...][None] + be_ref[...][None]

    Bb = 512
    g2 = ln_gamma.reshape(1, D)
    b2 = ln_beta.reshape(1, D)
    return pl.pallas_call(
        body,
        grid=(BATCH // Bb,),
        in_specs=[
            pl.BlockSpec((Bb, N_NUM), lambda i: (i, 0)),
            pl.BlockSpec((N_NUM, D), lambda i: (0, 0)),
            pl.BlockSpec((N_NUM, D), lambda i: (0, 0)),
            pl.BlockSpec((1, D), lambda i: (0, 0)),
            pl.BlockSpec((1, D), lambda i: (0, 0)),
        ],
        out_specs=pl.BlockSpec((Bb, N_NUM, D), lambda i: (i, 0, 0)),
        out_shape=jax.ShapeDtypeStruct((BATCH, N_NUM, D), jnp.float32),
    )(x_num, num_weight, num_bias, g2, b2)


def _sc_tokens(tab, xc2d, numtok):
    mesh = plsc.VectorSubcoreMesh(core_axis_name="c", subcore_axis_name="s")

    @functools.partial(
        pl.kernel,
        mesh=mesh,
        out_type=jax.ShapeDtypeStruct((BATCH * N_TOK, D), jnp.float32),
        scratch_types=[
            pltpu.VMEM((CAT_CHUNKS, CHUNK), jnp.int32),
            pltpu.VMEM((CAT_CHUNKS, CHUNK), jnp.int32),
            pltpu.VMEM((CAT_CHUNKS, CHUNK), jnp.int32),
            pltpu.VMEM((NUM_CHUNKS, CHUNK), jnp.int32),
            pltpu.VMEM((CHUNK, D), jnp.float32),
            pltpu.SemaphoreType.DMA,
        ],
        compiler_params=pltpu.CompilerParams(use_tc_tiling_on_sc=False),
    )
    def k(tab_hbm, xc_hbm, num_hbm, out_hbm, xcv, gidx, orow, onum, rows, sem):
        cid = lax.axis_index("c")
        sid = lax.axis_index("s")
        wid = sid * NC + cid
        b0 = wid * B_PER_W

        pltpu.sync_copy(xc_hbm.at[wid], xcv)

        def cat_idx_body(i, carry):
            c = i // (CHUNK // L)
            col = (i % (CHUNK // L)) * L
            ids = xcv[c, pl.ds(col, L)]
            j = i * L + lax.iota(jnp.int32, L)
            q = lax.div(j, N_CAT)
            f = j - q * N_CAT
            ids = jnp.minimum(jnp.maximum(ids, 0), CARD)
            gidx[c, pl.ds(col, L)] = f * (CARD + 1) + ids
            orow[c, pl.ds(col, L)] = (b0 + q) * N_TOK + N_NUM + f
            return carry

        lax.fori_loop(0, CAT_ROWS_W // L, cat_idx_body, 0)

        def num_idx_body(i, carry):
            c = i // (CHUNK // L)
            col = (i % (CHUNK // L)) * L
            j = i * L + lax.iota(jnp.int32, L)
            q = lax.div(j, N_NUM)
            f = j - q * N_NUM
            onum[c, pl.ds(col, L)] = (b0 + q) * N_TOK + f
            return carry

        lax.fori_loop(0, NUM_ROWS_W // L, num_idx_body, 0)

        def cat_dma_body(c, carry):
            pltpu.async_copy(tab_hbm.at[gidx.at[c]], rows, sem).wait()
            pltpu.async_copy(rows, out_hbm.at[orow.at[c]], sem).wait()
            return carry

        lax.fori_loop(0, CAT_CHUNKS, cat_dma_body, 0)

        def num_dma_body(c, carry):
            pltpu.sync_copy(num_hbm.at[pl.ds(b0 * N_NUM + c * CHUNK, CHUNK)], rows)
            pltpu.async_copy(rows, out_hbm.at[onum.at[c]], sem).wait()
            return carry

        lax.fori_loop(0, NUM_CHUNKS, num_dma_body, 0)

    return k(tab, xc2d, numtok)


def kernel(x_num, x_cat, num_weight, num_bias, ln_gamma, ln_beta, cat_tables):
    numtok = _num_tokens_tc(x_num, num_weight, num_bias, ln_gamma, ln_beta)
    tab = cat_tables.reshape(N_CAT * (CARD + 1), D)
    xc2d = x_cat.reshape(NW, CAT_CHUNKS, CHUNK)
    out = _sc_tokens(tab, xc2d, numtok.reshape(BATCH * N_NUM, D))
    return out.reshape(BATCH, N_TOK, D)
